# Initial kernel scaffold; baseline (speedup 1.0000x reference)
import jax
import jax.numpy as jnp
from jax.experimental import pallas as pl


def kernel(x, edge_index, W1, b1, W2, b2):
    raise NotImplementedError("write your pallas kernel here")


# speedup vs baseline: 2.8302x; 2.8302x over previous
import jax
import jax.numpy as jnp
from jax.experimental import pallas as pl


def _matmul_body(x_ref, w_ref, o_ref):
    o_ref[...] = jnp.dot(x_ref[...], w_ref[...],
                         preferred_element_type=jnp.float32)


def _mm(x, w, bm=1000):
    m, k = x.shape
    _, n = w.shape
    return pl.pallas_call(
        _matmul_body,
        grid=(m // bm,),
        in_specs=[pl.BlockSpec((bm, k), lambda i: (i, 0)),
                  pl.BlockSpec((k, n), lambda i: (0, 0))],
        out_specs=pl.BlockSpec((bm, n), lambda i: (i, 0)),
        out_shape=jax.ShapeDtypeStruct((m, n), jnp.float32),
    )(x, w)


def _gcn_layer(x, src, dst, dis, W, b):
    h = _mm(x, W)
    g = h * dis[:, None]
    seg = jnp.zeros_like(g).at[dst].add(g[src])
    return dis[:, None] * (seg + g) + b


def kernel(x, edge_index, W1, b1, W2, b2):
    n = x.shape[0]
    src = edge_index[0]
    dst = edge_index[1]
    deg = jnp.ones((n,), dtype=jnp.float32).at[dst].add(1.0)
    dis = jax.lax.rsqrt(deg)
    h = _gcn_layer(x, src, dst, dis, W1, b1)
    h = jax.nn.relu(h)
    h = _gcn_layer(h, src, dst, dis, W2, b2)
    return jax.nn.log_softmax(h, axis=1)


# speedup vs baseline: 11.1313x; 3.9331x over previous
import functools

import jax
import jax.numpy as jnp
from jax import lax
from jax.experimental import pallas as pl
from jax.experimental.pallas import tpu as pltpu
from jax.experimental.pallas import tpu_sc as plsc

N = 10000
E = 320000
NCH = 2560
E_PAD = NCH * 128
CPS1 = NCH // 16
CPS2 = NCH // 32
_mesh = plsc.VectorSubcoreMesh(core_axis_name="c", subcore_axis_name="s")


def _copy_tile_rows(src_ref, src_base, dst_ref, dst_base, sid):
    rb = sid * 632

    @pl.when(sid < 15)
    def _():
        pltpu.sync_copy(src_ref.at[pl.ds(src_base + rb, 632)],
                        dst_ref.at[pl.ds(dst_base + rb, 632)])

    @pl.when(sid == 15)
    def _():
        pltpu.sync_copy(src_ref.at[pl.ds(src_base + 9480, 520)],
                        dst_ref.at[pl.ds(dst_base + 9480, 520)])


HN = 10112
VPT = E_PAD // 32


@functools.partial(
    pl.kernel,
    out_type=jax.ShapeDtypeStruct((32, HN), jnp.float32),
    mesh=_mesh,
    scratch_types=[
        pltpu.VMEM((VPT,), jnp.int32),
        pltpu.VMEM((HN,), jnp.float32),
    ],
    compiler_params=pltpu.CompilerParams(needs_layout_passes=False),
)
def _deg_hist(dst_hbm, out_hbm, dstb, hist):
    cid = lax.axis_index("c")
    sid = lax.axis_index("s")
    tid = cid * 16 + sid
    pltpu.sync_copy(dst_hbm.at[pl.ds(tid * VPT, VPT)], dstb)
    zeros = jnp.zeros((16,), jnp.float32)

    def zbody(i, carry):
        hist[pl.ds(i * 16, 16)] = zeros
        return carry

    lax.fori_loop(0, HN // 16, zbody, 0)
    ones = jnp.ones((16,), jnp.float32)

    def body(i, carry):
        idx = dstb[pl.ds(i * 16, 16)]
        plsc.addupdate_scatter(hist, (idx,), ones)
        return carry

    lax.fori_loop(0, VPT // 16, body, 0)
    pltpu.sync_copy(hist, out_hbm.at[tid])


def _edge_sweep(g_hbm, srcb, dstb, rows0, rows1, acc, sem0, sem1, nch):
    pltpu.async_copy(g_hbm.at[srcb.at[0]], rows0, sem0)

    def body(jj, carry):
        j = 2 * jj
        pltpu.async_copy(g_hbm.at[srcb.at[j + 1]], rows1, sem1)
        pltpu.make_async_copy(g_hbm.at[srcb.at[j]], rows0, sem0).wait()
        pltpu.sync_copy(rows0, acc.at[dstb.at[j]], add=True)

        @pl.when(jj + 1 < nch // 2)
        def _():
            pltpu.async_copy(g_hbm.at[srcb.at[j + 2]], rows0, sem0)

        pltpu.make_async_copy(g_hbm.at[srcb.at[j + 1]], rows1, sem1).wait()
        pltpu.sync_copy(rows1, acc.at[dstb.at[j + 1]], add=True)
        return carry

    lax.fori_loop(0, nch // 2, body, 0)


@functools.partial(
    pl.kernel,
    out_type=jax.ShapeDtypeStruct((4 * N, 64), jnp.float32),
    mesh=_mesh,
    scratch_types=[
        pltpu.VMEM((CPS1, 128), jnp.int32),
        pltpu.VMEM((CPS1, 128), jnp.int32),
        pltpu.VMEM((128, 64), jnp.float32),
        pltpu.VMEM((128, 64), jnp.float32),
        pltpu.VMEM_SHARED((N + 8, 64), jnp.float32),
        pltpu.SemaphoreType.DMA,
        pltpu.SemaphoreType.DMA,
    ],
    compiler_params=pltpu.CompilerParams(use_tc_tiling_on_sc=False),
)
def _agg1(g_hbm, src_hbm, dst_hbm, out_hbm,
          srcb, dstb, rows0, rows1, acc, sem0, sem1):
    cid = lax.axis_index("c")
    sid = lax.axis_index("s")
    cb = sid * CPS1
    pltpu.sync_copy(dst_hbm.at[pl.ds(cb, CPS1)], dstb)
    for p in range(2):
        blk = 2 * cid + p
        pltpu.sync_copy(src_hbm.at[blk, pl.ds(cb, CPS1)], srcb)
        _copy_tile_rows(g_hbm, blk * N, acc, 0, sid)
        plsc.subcore_barrier()
        _edge_sweep(g_hbm, srcb, dstb, rows0, rows1, acc, sem0, sem1, CPS1)
        plsc.subcore_barrier()
        _copy_tile_rows(acc, 0, out_hbm, blk * N, sid)
        plsc.subcore_barrier()


@functools.partial(
    pl.kernel,
    out_type=jax.ShapeDtypeStruct((2 * N, 64), jnp.float32),
    mesh=_mesh,
    scratch_types=[
        pltpu.VMEM((CPS2, 128), jnp.int32),
        pltpu.VMEM((CPS2, 128), jnp.int32),
        pltpu.VMEM((128, 64), jnp.float32),
        pltpu.VMEM((128, 64), jnp.float32),
        pltpu.VMEM_SHARED((N + 8, 64), jnp.float32),
        pltpu.SemaphoreType.DMA,
        pltpu.SemaphoreType.DMA,
    ],
    compiler_params=pltpu.CompilerParams(use_tc_tiling_on_sc=False),
)
def _agg2(g_hbm, src_hbm, dst_hbm, out_hbm,
          srcb, dstb, rows0, rows1, acc, sem0, sem1):
    cid = lax.axis_index("c")
    sid = lax.axis_index("s")
    cb = (cid * 16 + sid) * CPS2
    pltpu.sync_copy(src_hbm.at[pl.ds(cb, CPS2)], srcb)
    pltpu.sync_copy(dst_hbm.at[pl.ds(cb, CPS2)], dstb)
    _copy_tile_rows(g_hbm, 0, acc, 0, sid)
    plsc.subcore_barrier()
    _edge_sweep(g_hbm, srcb, dstb, rows0, rows1, acc, sem0, sem1, CPS2)
    plsc.subcore_barrier()
    _copy_tile_rows(acc, 0, out_hbm, cid * N, sid)


_BM = 2000


def _mm1_body(x_ref, w_ref, dis_ref, o_ref):
    o_ref[...] = jnp.dot(x_ref[...], w_ref[0],
                         preferred_element_type=jnp.float32) * dis_ref[...]


def _mm1(x, w, dis):
    return pl.pallas_call(
        _mm1_body,
        grid=(4, N // _BM),
        in_specs=[
            pl.BlockSpec((_BM, 128), lambda f, i: (i, 0)),
            pl.BlockSpec((1, 128, 64), lambda f, i: (f, 0, 0)),
            pl.BlockSpec((_BM, 1), lambda f, i: (i, 0)),
        ],
        out_specs=pl.BlockSpec((_BM, 64), lambda f, i: (f * (N // _BM) + i, 0)),
        out_shape=jax.ShapeDtypeStruct((4 * N, 64), jnp.float32),
    )(x, w, dis)


def _mm2_body(a0_ref, a1_ref, a2_ref, a3_ref, dis_ref, b1_ref, w2_ref, o_ref):
    dis = dis_ref[...]
    acc = None
    for f, a_ref in enumerate((a0_ref, a1_ref, a2_ref, a3_ref)):
        xf = jnp.maximum(a_ref[...] * dis + b1_ref[f:f + 1, :], 0.0)
        d = jnp.dot(xf, w2_ref[64 * f:64 * (f + 1), :],
                    preferred_element_type=jnp.float32)
        acc = d if acc is None else acc + d
    o_ref[...] = acc * dis


def _mm2(out1, dis, b1r, w2):
    return pl.pallas_call(
        _mm2_body,
        grid=(N // _BM,),
        in_specs=[
            pl.BlockSpec((_BM, 64), lambda i, f=f: (f * (N // _BM) + i, 0))
            for f in range(4)
        ] + [
            pl.BlockSpec((_BM, 1), lambda i: (i, 0)),
            pl.BlockSpec((4, 64), lambda i: (0, 0)),
            pl.BlockSpec((256, 64), lambda i: (0, 0)),
        ],
        out_specs=pl.BlockSpec((_BM, 64), lambda i: (i, 0)),
        out_shape=jax.ShapeDtypeStruct((N, 64), jnp.float32),
    )(out1, out1, out1, out1, dis, b1r, w2)


def _final_body(p0_ref, p1_ref, g2_ref, dis_ref, b2_ref, o_ref):
    z = dis_ref[...] * (p0_ref[...] + p1_ref[...] - g2_ref[...]) + b2_ref[...]
    m = jnp.max(z, axis=1, keepdims=True)
    e = jnp.exp(z - m)
    s = jnp.sum(e, axis=1, keepdims=True)
    o_ref[...] = z - m - jnp.log(s)


def _final(p, g2, dis, b2r):
    return pl.pallas_call(
        _final_body,
        grid=(N // _BM,),
        in_specs=[
            pl.BlockSpec((_BM, 64), lambda i: (i, 0)),
            pl.BlockSpec((_BM, 64), lambda i: (N // _BM + i, 0)),
            pl.BlockSpec((_BM, 64), lambda i: (i, 0)),
            pl.BlockSpec((_BM, 1), lambda i: (i, 0)),
            pl.BlockSpec((1, 64), lambda i: (0, 0)),
        ],
        out_specs=pl.BlockSpec((_BM, 64), lambda i: (i, 0)),
        out_shape=jax.ShapeDtypeStruct((N, 64), jnp.float32),
    )(p, p, g2, dis, b2r)


def kernel(x, edge_index, W1, b1, W2, b2):
    src = edge_index[0]
    dst = edge_index[1]
    pad = E_PAD - E
    src_p = jnp.concatenate([src, jnp.zeros((pad,), jnp.int32)])
    dst_p = jnp.concatenate([dst, jnp.full((pad,), N, jnp.int32)])
    src0 = src_p.reshape(NCH, 128)
    src1 = jnp.stack([src0, src0 + N, src0 + 2 * N, src0 + 3 * N])
    dstc = dst_p.reshape(NCH, 128)
    hists = _deg_hist(dst_p)
    deg = 1.0 + jnp.sum(hists, axis=0)[:N]
    dis = jax.lax.rsqrt(deg)[:, None]

    W1r = W1.reshape(128, 4, 64).transpose(1, 0, 2)
    g1 = _mm1(x, W1r, dis)
    out1 = _agg1(g1, src1, dstc)
    g2 = _mm2(out1, dis, b1.reshape(4, 64), W2)
    p = _agg2(g2, src0, dstc)
    return _final(p, g2, dis, b2.reshape(1, 64))
